# baseline (speedup 1.0000x reference)
import functools

import jax
import jax.numpy as jnp
from jax import lax
from jax.experimental import pallas as pl
from jax.experimental.pallas import tpu as pltpu
from jax.experimental.pallas import tpu_sc as plsc

_NW = 32


def _make_mesh():
    return plsc.VectorSubcoreMesh(core_axis_name="c", subcore_axis_name="s")


def _make_reformat(V, D, W):
    V_main = (V // W) * W
    while (V - V_main) >= W:
        V_main += W
    n_win = V_main // W
    tail = V - V_main
    n_base, n_extra = divmod(n_win, _NW)
    mesh = _make_mesh()
    NC = mesh.num_cores
    n_groups = D // 16

    @functools.partial(
        pl.kernel,
        out_type=jax.ShapeDtypeStruct((V * D,), jnp.float32),
        mesh=mesh,
        compiler_params=pltpu.CompilerParams(
            use_tc_tiling_on_sc=True, needs_layout_passes=False
        ),
        scratch_types=[
            pltpu.VMEM((D, W), jnp.float32),
            pltpu.VMEM((D, W), jnp.float32),
            pltpu.VMEM((W * D,), jnp.float32),
            pltpu.VMEM((W * D,), jnp.float32),
            pltpu.SemaphoreType.DMA,
            pltpu.SemaphoreType.DMA,
            pltpu.SemaphoreType.DMA,
            pltpu.SemaphoreType.DMA,
        ],
    )
    def reformat(tt_hbm, aux_hbm, out_hbm, st_a, st_b, ob_a, ob_b,
                 si_a, si_b, so_a, so_b):
        wid = lax.axis_index("s") * NC + lax.axis_index("c")
        n_my = n_base + jnp.where(wid < n_extra, 1, 0)

        def w0_of(k):
            return (k * _NW + wid) * W

        def start_in(k, st, si):
            w0 = w0_of(k)
            for i in range(D // 8):
                pltpu.async_copy(
                    tt_hbm.at[pl.ds(8 * i, 8), pl.ds(w0, W)],
                    st.at[pl.ds(8 * i, 8), pl.ds(0, W)],
                    si,
                )

        def wait_in(st, si):
            for i in range(D // 8):
                pltpu.make_async_copy(
                    tt_hbm.at[pl.ds(8 * i, 8), pl.ds(0, W)],
                    st.at[pl.ds(8 * i, 8), pl.ds(0, W)],
                    si,
                ).wait()

        def wait_out(ob, so):
            pltpu.make_async_copy(ob, out_hbm.at[pl.ds(0, W * D)], so).wait()

        rows16 = lax.iota(jnp.int32, 16)

        def transpose_win(st, ob):
            def col_body(c0, _):
                for u in range(2):
                    c = c0 * 2 + u
                    cols = jnp.full((16,), c, jnp.int32)
                    for g in range(n_groups):
                        v = plsc.load_gather(st, [rows16 + 16 * g, cols])
                        ob[pl.ds(c * D + 16 * g, 16)] = v
                return 0

            lax.fori_loop(0, W // 2, col_body, 0)

        def step(k, st, ob, si, so):
            wait_in(st, si)

            @pl.when(k >= 2)
            def _():
                wait_out(ob, so)

            transpose_win(st, ob)
            pltpu.async_copy(ob, out_hbm.at[pl.ds(w0_of(k) * D, W * D)], so)

        if tail:
            @pl.when(wid == 0)
            def _():
                pltpu.sync_copy(aux_hbm, ob_a.at[pl.ds(0, tail * D)])
                pltpu.sync_copy(ob_a.at[pl.ds(0, tail * D)],
                                out_hbm.at[pl.ds(V_main * D, tail * D)])

        start_in(0, st_a, si_a)

        def pair_body(i, _):
            k0 = 2 * i
            k1 = k0 + 1

            @pl.when(k1 < n_my)
            def _():
                start_in(k1, st_b, si_b)

            step(k0, st_a, ob_a, si_a, so_a)

            @pl.when(k1 < n_my)
            def _():
                @pl.when(k1 + 1 < n_my)
                def _():
                    start_in(k1 + 1, st_a, si_a)

                step(k1, st_b, ob_b, si_b, so_b)

            return 0

        lax.fori_loop(0, (n_my + 1) // 2, pair_body, 0)
        wait_out(ob_a, so_a)

        @pl.when(n_my >= 2)
        def _():
            wait_out(ob_b, so_b)

    return reformat


def _chunks_of_L(L):
    chunks = []
    off = 0
    while off < L:
        size = min(128, L - off)
        if L - off > 128:
            size -= size % 8
        chunks.append((off, size))
        off += size
    return chunks


def _make_sc_sum(B, L, V, D):
    assert B % _NW == 0
    dpw = B // _NW
    assert (dpw * L) % 8 == 0 and (L % 8) == 0
    chunks = _chunks_of_L(L)
    n_groups = D // 16
    mesh = _make_mesh()
    NC = mesh.num_cores

    @functools.partial(
        pl.kernel,
        out_type=jax.ShapeDtypeStruct((B, D), jnp.float32),
        mesh=mesh,
        compiler_params=pltpu.CompilerParams(use_tc_tiling_on_sc=False),
        scratch_types=[
            pltpu.VMEM((dpw * L,), jnp.int32),
            pltpu.VMEM((L, D), jnp.float32),
            pltpu.VMEM((L, D), jnp.float32),
            pltpu.VMEM((dpw, D), jnp.float32),
            pltpu.SemaphoreType.DMA,
            pltpu.SemaphoreType.DMA,
        ],
    )
    def sc_sum(tokens_hbm, table_hbm, out_hbm, idx_v, buf_a, buf_b, outblk,
               sem_a, sem_b):
        wid = lax.axis_index("s") * NC + lax.axis_index("c")
        base_doc = wid * dpw

        pltpu.sync_copy(tokens_hbm.at[pl.ds(base_doc * L, dpw * L)], idx_v)

        def gather_start(d, buf, sem):
            off = d * L
            for c_off, c_sz in chunks:
                pltpu.async_copy(
                    table_hbm.at[idx_v.at[pl.ds(off + c_off, c_sz)]],
                    buf.at[pl.ds(c_off, c_sz)],
                    sem,
                )

        def gather_wait(buf, sem):
            for c_off, c_sz in chunks:
                pltpu.make_async_copy(
                    table_hbm.at[idx_v.at[pl.ds(c_off, c_sz)]],
                    buf.at[pl.ds(c_off, c_sz)],
                    sem,
                ).wait()

        UNROLL = 8
        assert L % UNROLL == 0

        def reduce_doc(buf, d):
            zero = jnp.zeros((16,), jnp.float32)

            def body(t0, accs):
                accs = list(accs)
                for j in range(UNROLL):
                    t = t0 * UNROLL + j
                    for g in range(n_groups):
                        accs[g] = accs[g] + buf[t, pl.ds(g * 16, 16)]
                return tuple(accs)

            accs = lax.fori_loop(0, L // UNROLL, body, (zero,) * n_groups)
            for g in range(n_groups):
                outblk[d, pl.ds(g * 16, 16)] = accs[g]

        gather_start(0, buf_a, sem_a)

        def pair_body(i, _):
            d0 = 2 * i
            gather_start(d0 + 1, buf_b, sem_b)
            gather_wait(buf_a, sem_a)
            reduce_doc(buf_a, d0)

            @pl.when(d0 + 2 < dpw)
            def _():
                gather_start(d0 + 2, buf_a, sem_a)

            gather_wait(buf_b, sem_b)
            reduce_doc(buf_b, d0 + 1)
            return 0

        lax.fori_loop(0, dpw // 2, pair_body, 0)

        pltpu.sync_copy(outblk, out_hbm.at[pl.ds(base_doc, dpw)])

    return sc_sum


def _linear_body(sums_ref, len_ref, w_ref, b_ref, out_ref):
    inv = 1.0 / jnp.maximum(len_ref[...], 1).astype(jnp.float32)
    emb = sums_ref[...] * inv
    out_ref[...] = (
        jnp.dot(emb, w_ref[...], preferred_element_type=jnp.float32) + b_ref[...]
    )


def _tc_linear(sums, lens2d, W, b2d, BLK=512):
    B, D = sums.shape
    NL = W.shape[1]
    return pl.pallas_call(
        _linear_body,
        grid=(B // BLK,),
        in_specs=[
            pl.BlockSpec((BLK, D), lambda i: (i, 0)),
            pl.BlockSpec((BLK, 1), lambda i: (i, 0)),
            pl.BlockSpec((D, NL), lambda i: (0, 0)),
            pl.BlockSpec((1, NL), lambda i: (0, 0)),
        ],
        out_specs=pl.BlockSpec((BLK, NL), lambda i: (i, 0)),
        out_shape=jax.ShapeDtypeStruct((B, NL), jnp.float32),
    )(sums, lens2d, W, b2d)


@jax.jit
def kernel(doc_token, doc_token_len, token_table, W, b):
    B, L = doc_token.shape
    V, D = token_table.shape
    WIN = 256
    V_main = (V // WIN) * WIN
    reformat = _make_reformat(V, D, WIN)
    sc_sum = _make_sc_sum(B, L, V, D)

    tt = token_table.T
    aux = token_table[V_main:].reshape(-1)
    table_flat = reformat(tt, aux)
    table_rm = table_flat.reshape(V, D)

    tokens = doc_token.reshape(-1).astype(jnp.int32)
    sums = sc_sum(tokens, table_rm)
    lens2d = doc_token_len.reshape(B, 1)
    b2d = b.reshape(1, -1)
    return _tc_linear(sums, lens2d, W, b2d)


# speedup vs baseline: 2.4568x; 2.4568x over previous
import functools

import jax
import jax.numpy as jnp
from jax import lax
from jax.experimental import pallas as pl
from jax.experimental.pallas import tpu as pltpu
from jax.experimental.pallas import tpu_sc as plsc

_NW = 32


def _make_mesh():
    return plsc.VectorSubcoreMesh(core_axis_name="c", subcore_axis_name="s")


def _transpose_body(in_ref, out_ref):
    t = in_ref[...].T
    out_ref[...] = jnp.concatenate([t, t], axis=1)


def _tc_transpose(tt, V, D, BLKV=2048):
    grid = (V + BLKV - 1) // BLKV
    return pl.pallas_call(
        _transpose_body,
        grid=(grid,),
        in_specs=[pl.BlockSpec((D, BLKV), lambda i: (0, i))],
        out_specs=pl.BlockSpec((BLKV, 2 * D), lambda i: (i, 0)),
        out_shape=jax.ShapeDtypeStruct((V, 2 * D), jnp.float32),
    )(tt)


def _chunks_of_L(L):
    chunks = []
    off = 0
    while off < L:
        size = min(128, L - off)
        if L - off > 128:
            size -= size % 8
        chunks.append((off, size))
        off += size
    return chunks


def _make_sc_sum(B, L, V, D):
    assert B % _NW == 0
    dpw = B // _NW
    assert (dpw * L) % 8 == 0 and (L % 8) == 0
    chunks = _chunks_of_L(L)
    n_groups = D // 16
    mesh = _make_mesh()
    NC = mesh.num_cores

    @functools.partial(
        pl.kernel,
        out_type=jax.ShapeDtypeStruct((B, D), jnp.float32),
        mesh=mesh,
        compiler_params=pltpu.CompilerParams(use_tc_tiling_on_sc=False),
        scratch_types=[
            pltpu.VMEM((dpw * L,), jnp.int32),
            pltpu.VMEM((L, D), jnp.float32),
            pltpu.VMEM((L, D), jnp.float32),
            pltpu.VMEM((dpw, D), jnp.float32),
            pltpu.SemaphoreType.DMA,
            pltpu.SemaphoreType.DMA,
        ],
    )
    def sc_sum(tokens_hbm, table_hbm, out_hbm, idx_v, buf_a, buf_b, outblk,
               sem_a, sem_b):
        wid = lax.axis_index("s") * NC + lax.axis_index("c")
        base_doc = wid * dpw

        pltpu.sync_copy(tokens_hbm.at[pl.ds(base_doc * L, dpw * L)], idx_v)

        def gather_start(d, buf, sem):
            off = d * L
            for c_off, c_sz in chunks:
                pltpu.async_copy(
                    table_hbm.at[idx_v.at[pl.ds(off + c_off, c_sz)]],
                    buf.at[pl.ds(c_off, c_sz)],
                    sem,
                )

        def gather_wait(buf, sem):
            for c_off, c_sz in chunks:
                pltpu.make_async_copy(
                    table_hbm.at[idx_v.at[pl.ds(c_off, c_sz)]],
                    buf.at[pl.ds(c_off, c_sz)],
                    sem,
                ).wait()

        UNROLL = 8
        assert L % UNROLL == 0

        def reduce_doc(buf, d):
            zero = jnp.zeros((16,), jnp.float32)

            def body(t0, accs):
                accs = list(accs)
                for j in range(UNROLL):
                    t = t0 * UNROLL + j
                    for g in range(n_groups):
                        accs[g] = accs[g] + buf[t, pl.ds(g * 16, 16)]
                return tuple(accs)

            accs = lax.fori_loop(0, L // UNROLL, body, (zero,) * n_groups)
            for g in range(n_groups):
                outblk[d, pl.ds(g * 16, 16)] = accs[g]

        gather_start(0, buf_a, sem_a)

        def pair_body(i, _):
            d0 = 2 * i
            gather_start(d0 + 1, buf_b, sem_b)
            gather_wait(buf_a, sem_a)
            reduce_doc(buf_a, d0)

            @pl.when(d0 + 2 < dpw)
            def _():
                gather_start(d0 + 2, buf_a, sem_a)

            gather_wait(buf_b, sem_b)
            reduce_doc(buf_b, d0 + 1)
            return 0

        lax.fori_loop(0, dpw // 2, pair_body, 0)

        pltpu.sync_copy(outblk, out_hbm.at[pl.ds(base_doc, dpw)])

    return sc_sum


def _linear_body(sums_ref, len_ref, w_ref, b_ref, out_ref):
    inv = 1.0 / jnp.maximum(len_ref[...], 1).astype(jnp.float32)
    emb = sums_ref[...] * inv
    out_ref[...] = (
        jnp.dot(emb, w_ref[...], preferred_element_type=jnp.float32) + b_ref[...]
    )


def _tc_linear(sums, lens2d, W, b2d, BLK=512):
    B, D = sums.shape
    NL = W.shape[1]
    return pl.pallas_call(
        _linear_body,
        grid=(B // BLK,),
        in_specs=[
            pl.BlockSpec((BLK, D), lambda i: (i, 0)),
            pl.BlockSpec((BLK, 1), lambda i: (i, 0)),
            pl.BlockSpec((D, NL), lambda i: (0, 0)),
            pl.BlockSpec((1, NL), lambda i: (0, 0)),
        ],
        out_specs=pl.BlockSpec((BLK, NL), lambda i: (i, 0)),
        out_shape=jax.ShapeDtypeStruct((B, NL), jnp.float32),
    )(sums, lens2d, W, b2d)


@jax.jit
def kernel(doc_token, doc_token_len, token_table, W, b):
    B, L = doc_token.shape
    V, D = token_table.shape
    sc_sum = _make_sc_sum(B, L, V, D)

    tt = token_table.T
    table2 = _tc_transpose(tt, V, D)
    table_rm = table2.reshape(2 * V, D)

    tokens = doc_token.reshape(-1).astype(jnp.int32) * 2
    sums = sc_sum(tokens, table_rm)
    lens2d = doc_token_len.reshape(B, 1)
    b2d = b.reshape(1, -1)
    return _tc_linear(sums, lens2d, W, b2d)


# speedup vs baseline: 3.3646x; 1.3695x over previous
import functools

import jax
import jax.numpy as jnp
from jax import lax
from jax.experimental import pallas as pl
from jax.experimental.pallas import tpu as pltpu
from jax.experimental.pallas import tpu_sc as plsc

_NW = 32


def _make_mesh():
    return plsc.VectorSubcoreMesh(core_axis_name="c", subcore_axis_name="s")


def _transpose_body(in_ref, out_ref):
    D = in_ref.shape[0]
    out_ref[:, 0:D] = in_ref[...].T


def _tc_transpose(tt, V, D, BLKV=4096):
    grid = (V + BLKV - 1) // BLKV
    return pl.pallas_call(
        _transpose_body,
        grid=(grid,),
        in_specs=[pl.BlockSpec((D, BLKV), lambda i: (0, i))],
        out_specs=pl.BlockSpec((BLKV, 2 * D), lambda i: (i, 0)),
        out_shape=jax.ShapeDtypeStruct((V, 2 * D), jnp.float32),
        compiler_params=pltpu.CompilerParams(
            dimension_semantics=("parallel",)
        ),
    )(tt)


def _chunks_of_L(L):
    chunks = []
    off = 0
    while off < L:
        size = min(128, L - off)
        if L - off > 128:
            size -= size % 8
        chunks.append((off, size))
        off += size
    return chunks


def _make_sc_sum(B, L, V, D):
    assert B % _NW == 0
    dpw = B // _NW
    assert (dpw * L) % 8 == 0 and (L % 8) == 0
    chunks = _chunks_of_L(L)
    n_groups = D // 16
    mesh = _make_mesh()
    NC = mesh.num_cores

    @functools.partial(
        pl.kernel,
        out_type=jax.ShapeDtypeStruct((B, D), jnp.float32),
        mesh=mesh,
        compiler_params=pltpu.CompilerParams(use_tc_tiling_on_sc=False),
        scratch_types=[
            pltpu.VMEM((dpw * L,), jnp.int32),
            pltpu.VMEM((L, D), jnp.float32),
            pltpu.VMEM((L, D), jnp.float32),
            pltpu.VMEM((dpw, D), jnp.float32),
            pltpu.SemaphoreType.DMA,
            pltpu.SemaphoreType.DMA,
        ],
    )
    def sc_sum(tokens_hbm, table_hbm, out_hbm, idx_v, buf_a, buf_b, outblk,
               sem_a, sem_b):
        wid = lax.axis_index("s") * NC + lax.axis_index("c")
        base_doc = wid * dpw

        pltpu.sync_copy(tokens_hbm.at[pl.ds(base_doc * L, dpw * L)], idx_v)

        def gather_start(d, buf, sem):
            off = d * L
            for c_off, c_sz in chunks:
                pltpu.async_copy(
                    table_hbm.at[idx_v.at[pl.ds(off + c_off, c_sz)]],
                    buf.at[pl.ds(c_off, c_sz)],
                    sem,
                )

        def gather_wait(buf, sem):
            for c_off, c_sz in chunks:
                pltpu.make_async_copy(
                    table_hbm.at[idx_v.at[pl.ds(c_off, c_sz)]],
                    buf.at[pl.ds(c_off, c_sz)],
                    sem,
                ).wait()

        UNROLL = 8
        assert L % UNROLL == 0

        def reduce_doc(buf, d):
            zero = jnp.zeros((16,), jnp.float32)

            def body(t0, accs):
                accs = list(accs)
                for j in range(UNROLL):
                    t = t0 * UNROLL + j
                    for g in range(n_groups):
                        accs[g] = accs[g] + buf[t, pl.ds(g * 16, 16)]
                return tuple(accs)

            accs = lax.fori_loop(0, L // UNROLL, body, (zero,) * n_groups)
            for g in range(n_groups):
                outblk[d, pl.ds(g * 16, 16)] = accs[g]

        gather_start(0, buf_a, sem_a)

        def pair_body(i, _):
            d0 = 2 * i
            gather_start(d0 + 1, buf_b, sem_b)
            gather_wait(buf_a, sem_a)
            reduce_doc(buf_a, d0)

            @pl.when(d0 + 2 < dpw)
            def _():
                gather_start(d0 + 2, buf_a, sem_a)

            gather_wait(buf_b, sem_b)
            reduce_doc(buf_b, d0 + 1)
            return 0

        lax.fori_loop(0, dpw // 2, pair_body, 0)

        pltpu.sync_copy(outblk, out_hbm.at[pl.ds(base_doc, dpw)])

    return sc_sum


def _linear_body(sums_ref, len_ref, w_ref, b_ref, out_ref):
    inv = 1.0 / jnp.maximum(len_ref[...], 1).astype(jnp.float32)
    emb = sums_ref[...] * inv
    out_ref[...] = (
        jnp.dot(emb, w_ref[...], preferred_element_type=jnp.float32) + b_ref[...]
    )


def _tc_linear(sums, lens2d, W, b2d, BLK=512):
    B, D = sums.shape
    NL = W.shape[1]
    return pl.pallas_call(
        _linear_body,
        grid=(B // BLK,),
        in_specs=[
            pl.BlockSpec((BLK, D), lambda i: (i, 0)),
            pl.BlockSpec((BLK, 1), lambda i: (i, 0)),
            pl.BlockSpec((D, NL), lambda i: (0, 0)),
            pl.BlockSpec((1, NL), lambda i: (0, 0)),
        ],
        out_specs=pl.BlockSpec((BLK, NL), lambda i: (i, 0)),
        out_shape=jax.ShapeDtypeStruct((B, NL), jnp.float32),
    )(sums, lens2d, W, b2d)


@jax.jit
def kernel(doc_token, doc_token_len, token_table, W, b):
    B, L = doc_token.shape
    V, D = token_table.shape
    sc_sum = _make_sc_sum(B, L, V, D)

    tt = token_table.T
    table2 = _tc_transpose(tt, V, D)
    table_rm = table2.reshape(2 * V, D)

    tokens = doc_token.reshape(-1).astype(jnp.int32) * 2
    sums = sc_sum(tokens, table_rm)
    lens2d = doc_token_len.reshape(B, 1)
    b2d = b.reshape(1, -1)
    return _tc_linear(sums, lens2d, W, b2d)


# speedup vs baseline: 3.9510x; 1.1743x over previous
import functools

import jax
import jax.numpy as jnp
from jax import lax
from jax.experimental import pallas as pl
from jax.experimental.pallas import tpu as pltpu
from jax.experimental.pallas import tpu_sc as plsc

_NW = 32


def _make_mesh():
    return plsc.VectorSubcoreMesh(core_axis_name="c", subcore_axis_name="s")


def _transpose_body(a_ref, b_ref, out_ref):
    D = a_ref.shape[0]
    out_ref[:, 0:D] = a_ref[...].T
    out_ref[:, D:2 * D] = b_ref[...].T


def _tc_transpose(tt, V, D, H, BLKV=4096):
    grid = H // BLKV
    last_blk = (V - 1) // BLKV

    return pl.pallas_call(
        _transpose_body,
        grid=(grid,),
        in_specs=[
            pl.BlockSpec((D, BLKV), lambda i: (0, i)),
            pl.BlockSpec((D, BLKV), lambda i: (0, jnp.minimum(grid + i, last_blk))),
        ],
        out_specs=pl.BlockSpec((BLKV, 2 * D), lambda i: (i, 0)),
        out_shape=jax.ShapeDtypeStruct((H, 2 * D), jnp.float32),
        compiler_params=pltpu.CompilerParams(
            dimension_semantics=("parallel",)
        ),
    )(tt, tt)


def _chunks_of_L(L):
    chunks = []
    off = 0
    while off < L:
        size = min(128, L - off)
        if L - off > 128:
            size -= size % 8
        chunks.append((off, size))
        off += size
    return chunks


def _make_sc_sum(B, L, V, D):
    assert B % _NW == 0
    dpw = B // _NW
    assert (dpw * L) % 8 == 0 and (L % 8) == 0
    chunks = _chunks_of_L(L)
    n_groups = D // 16
    mesh = _make_mesh()
    NC = mesh.num_cores

    @functools.partial(
        pl.kernel,
        out_type=jax.ShapeDtypeStruct((B, D), jnp.float32),
        mesh=mesh,
        compiler_params=pltpu.CompilerParams(use_tc_tiling_on_sc=False),
        scratch_types=[
            pltpu.VMEM((dpw * L,), jnp.int32),
            pltpu.VMEM((L, D), jnp.float32),
            pltpu.VMEM((L, D), jnp.float32),
            pltpu.VMEM((dpw, D), jnp.float32),
            pltpu.SemaphoreType.DMA,
            pltpu.SemaphoreType.DMA,
        ],
    )
    def sc_sum(tokens_hbm, table_hbm, out_hbm, idx_v, buf_a, buf_b, outblk,
               sem_a, sem_b):
        wid = lax.axis_index("s") * NC + lax.axis_index("c")
        base_doc = wid * dpw

        pltpu.sync_copy(tokens_hbm.at[pl.ds(base_doc * L, dpw * L)], idx_v)

        def gather_start(d, buf, sem):
            off = d * L
            for c_off, c_sz in chunks:
                pltpu.async_copy(
                    table_hbm.at[idx_v.at[pl.ds(off + c_off, c_sz)]],
                    buf.at[pl.ds(c_off, c_sz)],
                    sem,
                )

        def gather_wait(buf, sem):
            for c_off, c_sz in chunks:
                pltpu.make_async_copy(
                    table_hbm.at[idx_v.at[pl.ds(c_off, c_sz)]],
                    buf.at[pl.ds(c_off, c_sz)],
                    sem,
                ).wait()

        UNROLL = 8
        assert L % UNROLL == 0

        def reduce_doc(buf, d):
            zero = jnp.zeros((16,), jnp.float32)

            def body(t0, accs):
                accs = list(accs)
                for j in range(UNROLL):
                    t = t0 * UNROLL + j
                    for g in range(n_groups):
                        accs[g] = accs[g] + buf[t, pl.ds(g * 16, 16)]
                return tuple(accs)

            accs = lax.fori_loop(0, L // UNROLL, body, (zero,) * n_groups)
            for g in range(n_groups):
                outblk[d, pl.ds(g * 16, 16)] = accs[g]

        gather_start(0, buf_a, sem_a)

        def pair_body(i, _):
            d0 = 2 * i
            gather_start(d0 + 1, buf_b, sem_b)
            gather_wait(buf_a, sem_a)
            reduce_doc(buf_a, d0)

            @pl.when(d0 + 2 < dpw)
            def _():
                gather_start(d0 + 2, buf_a, sem_a)

            gather_wait(buf_b, sem_b)
            reduce_doc(buf_b, d0 + 1)
            return 0

        lax.fori_loop(0, dpw // 2, pair_body, 0)

        pltpu.sync_copy(outblk, out_hbm.at[pl.ds(base_doc, dpw)])

    return sc_sum


def _linear_body(sums_ref, len_ref, w_ref, b_ref, out_ref):
    inv = 1.0 / jnp.maximum(len_ref[...], 1).astype(jnp.float32)
    emb = sums_ref[...] * inv
    out_ref[...] = (
        jnp.dot(emb, w_ref[...], preferred_element_type=jnp.float32) + b_ref[...]
    )


def _tc_linear(sums, lens2d, W, b2d, BLK=512):
    B, D = sums.shape
    NL = W.shape[1]
    return pl.pallas_call(
        _linear_body,
        grid=(B // BLK,),
        in_specs=[
            pl.BlockSpec((BLK, D), lambda i: (i, 0)),
            pl.BlockSpec((BLK, 1), lambda i: (i, 0)),
            pl.BlockSpec((D, NL), lambda i: (0, 0)),
            pl.BlockSpec((1, NL), lambda i: (0, 0)),
        ],
        out_specs=pl.BlockSpec((BLK, NL), lambda i: (i, 0)),
        out_shape=jax.ShapeDtypeStruct((B, NL), jnp.float32),
    )(sums, lens2d, W, b2d)


@jax.jit
def kernel(doc_token, doc_token_len, token_table, W, b):
    B, L = doc_token.shape
    V, D = token_table.shape
    sc_sum = _make_sc_sum(B, L, V, D)

    H = 524288
    tt = token_table.T
    table2 = _tc_transpose(tt, V, D, H)
    table_rm = table2.reshape(2 * H, D)

    tok = doc_token.reshape(-1).astype(jnp.int32)
    tokens = jnp.where(tok < H, 2 * tok, 2 * (tok - H) + 1)
    sums = sc_sum(tokens, table_rm)
    lens2d = doc_token_len.reshape(B, 1)
    b2d = b.reshape(1, -1)
    return _tc_linear(sums, lens2d, W, b2d)


# speedup vs baseline: 4.3110x; 1.0911x over previous
import functools

import jax
import jax.numpy as jnp
from jax import lax
from jax.experimental import pallas as pl
from jax.experimental.pallas import tpu as pltpu
from jax.experimental.pallas import tpu_sc as plsc

_NW = 32


def _make_mesh():
    return plsc.VectorSubcoreMesh(core_axis_name="c", subcore_axis_name="s")


def _transpose_body(a_ref, b_ref, out_ref):
    D = a_ref.shape[0]
    out_ref[:, 0:D] = a_ref[...].T
    out_ref[:, D:2 * D] = b_ref[...].T


def _tc_transpose(tt, V, D, H, BLKV=8192):
    grid = H // BLKV
    last_blk = (V - 1) // BLKV

    return pl.pallas_call(
        _transpose_body,
        grid=(grid,),
        in_specs=[
            pl.BlockSpec((D, BLKV), lambda i: (0, i)),
            pl.BlockSpec((D, BLKV), lambda i: (0, jnp.minimum(grid + i, last_blk))),
        ],
        out_specs=pl.BlockSpec((BLKV, 2 * D), lambda i: (i, 0)),
        out_shape=jax.ShapeDtypeStruct((H, 2 * D), jnp.float32),
        compiler_params=pltpu.CompilerParams(
            dimension_semantics=("parallel",)
        ),
    )(tt, tt)


def _chunks_of_L(L):
    chunks = []
    off = 0
    while off < L:
        size = min(128, L - off)
        if L - off > 128:
            size -= size % 8
        chunks.append((off, size))
        off += size
    return chunks


def _make_sc_sum(B, L, V, D):
    assert B % _NW == 0
    dpw = B // _NW
    assert (dpw * L) % 8 == 0 and (L % 8) == 0
    chunks = _chunks_of_L(L)
    n_groups = D // 16
    mesh = _make_mesh()
    NC = mesh.num_cores

    @functools.partial(
        pl.kernel,
        out_type=jax.ShapeDtypeStruct((B, D), jnp.float32),
        mesh=mesh,
        compiler_params=pltpu.CompilerParams(use_tc_tiling_on_sc=False),
        scratch_types=[
            pltpu.VMEM((dpw * L,), jnp.int32),
            pltpu.VMEM((L, D), jnp.float32),
            pltpu.VMEM((L, D), jnp.float32),
            pltpu.VMEM((dpw, D), jnp.float32),
            pltpu.SemaphoreType.DMA,
            pltpu.SemaphoreType.DMA,
        ],
    )
    def sc_sum(tokens_hbm, table_hbm, out_hbm, idx_v, buf_a, buf_b, outblk,
               sem_a, sem_b):
        wid = lax.axis_index("s") * NC + lax.axis_index("c")
        base_doc = wid * dpw

        pltpu.sync_copy(tokens_hbm.at[pl.ds(base_doc * L, dpw * L)], idx_v)

        def gather_start(d, buf, sem):
            off = d * L
            for c_off, c_sz in chunks:
                pltpu.async_copy(
                    table_hbm.at[idx_v.at[pl.ds(off + c_off, c_sz)]],
                    buf.at[pl.ds(c_off, c_sz)],
                    sem,
                )

        def gather_wait(buf, sem):
            for c_off, c_sz in chunks:
                pltpu.make_async_copy(
                    table_hbm.at[idx_v.at[pl.ds(c_off, c_sz)]],
                    buf.at[pl.ds(c_off, c_sz)],
                    sem,
                ).wait()

        UNROLL = 8
        assert L % UNROLL == 0

        def reduce_doc(buf, d):
            zero = jnp.zeros((16,), jnp.float32)

            def body(t0, accs):
                accs = list(accs)
                for j in range(UNROLL):
                    t = t0 * UNROLL + j
                    for g in range(n_groups):
                        accs[g] = accs[g] + buf[t, pl.ds(g * 16, 16)]
                return tuple(accs)

            accs = lax.fori_loop(0, L // UNROLL, body, (zero,) * n_groups)
            for g in range(n_groups):
                outblk[d, pl.ds(g * 16, 16)] = accs[g]

        gather_start(0, buf_a, sem_a)

        def pair_body(i, _):
            d0 = 2 * i
            gather_start(d0 + 1, buf_b, sem_b)
            gather_wait(buf_a, sem_a)
            reduce_doc(buf_a, d0)

            @pl.when(d0 + 2 < dpw)
            def _():
                gather_start(d0 + 2, buf_a, sem_a)

            gather_wait(buf_b, sem_b)
            reduce_doc(buf_b, d0 + 1)
            return 0

        lax.fori_loop(0, dpw // 2, pair_body, 0)

        pltpu.sync_copy(outblk, out_hbm.at[pl.ds(base_doc, dpw)])

    return sc_sum


def _linear_body(sums_ref, len_ref, w_ref, b_ref, out_ref):
    inv = 1.0 / jnp.maximum(len_ref[...], 1).astype(jnp.float32)
    emb = sums_ref[...] * inv
    out_ref[...] = (
        jnp.dot(emb, w_ref[...], preferred_element_type=jnp.float32) + b_ref[...]
    )


def _tc_linear(sums, lens2d, W, b2d, BLK=512):
    B, D = sums.shape
    NL = W.shape[1]
    return pl.pallas_call(
        _linear_body,
        grid=(B // BLK,),
        in_specs=[
            pl.BlockSpec((BLK, D), lambda i: (i, 0)),
            pl.BlockSpec((BLK, 1), lambda i: (i, 0)),
            pl.BlockSpec((D, NL), lambda i: (0, 0)),
            pl.BlockSpec((1, NL), lambda i: (0, 0)),
        ],
        out_specs=pl.BlockSpec((BLK, NL), lambda i: (i, 0)),
        out_shape=jax.ShapeDtypeStruct((B, NL), jnp.float32),
    )(sums, lens2d, W, b2d)


@jax.jit
def kernel(doc_token, doc_token_len, token_table, W, b):
    B, L = doc_token.shape
    V, D = token_table.shape
    sc_sum = _make_sc_sum(B, L, V, D)

    H = 524288
    tt = token_table.T
    table2 = _tc_transpose(tt, V, D, H)
    table_rm = table2.reshape(2 * H, D)

    tok = doc_token.reshape(-1).astype(jnp.int32)
    tokens = jnp.where(tok < H, 2 * tok, 2 * (tok - H) + 1)
    sums = sc_sum(tokens, table_rm)
    lens2d = doc_token_len.reshape(B, 1)
    b2d = b.reshape(1, -1)
    return _tc_linear(sums, lens2d, W, b2d)


# speedup vs baseline: 4.4794x; 1.0391x over previous
import functools

import jax
import jax.numpy as jnp
from jax import lax
from jax.experimental import pallas as pl
from jax.experimental.pallas import tpu as pltpu
from jax.experimental.pallas import tpu_sc as plsc

_NW = 32


def _make_mesh():
    return plsc.VectorSubcoreMesh(core_axis_name="c", subcore_axis_name="s")


def _transpose_body(a_ref, b_ref, out_ref):
    D = a_ref.shape[0]
    out_ref[:, 0:D] = a_ref[...].T
    out_ref[:, D:2 * D] = b_ref[...].T


def _tc_transpose(tt, V, D, H, BLKV=16384):
    grid = H // BLKV
    last_blk = (V - 1) // BLKV

    return pl.pallas_call(
        _transpose_body,
        grid=(grid,),
        in_specs=[
            pl.BlockSpec((D, BLKV), lambda i: (0, i)),
            pl.BlockSpec((D, BLKV), lambda i: (0, jnp.minimum(grid + i, last_blk))),
        ],
        out_specs=pl.BlockSpec((BLKV, 2 * D), lambda i: (i, 0)),
        out_shape=jax.ShapeDtypeStruct((H, 2 * D), jnp.float32),
        compiler_params=pltpu.CompilerParams(
            dimension_semantics=("parallel",)
        ),
    )(tt, tt)


def _chunks_of_L(L):
    chunks = []
    off = 0
    while off < L:
        size = min(128, L - off)
        if L - off > 128:
            size -= size % 8
        chunks.append((off, size))
        off += size
    return chunks


def _make_sc_sum(B, L, V, D):
    assert B % _NW == 0
    dpw = B // _NW
    assert (dpw * L) % 8 == 0 and (L % 8) == 0
    chunks = _chunks_of_L(L)
    n_groups = D // 16
    mesh = _make_mesh()
    NC = mesh.num_cores

    @functools.partial(
        pl.kernel,
        out_type=jax.ShapeDtypeStruct((B, D), jnp.float32),
        mesh=mesh,
        compiler_params=pltpu.CompilerParams(use_tc_tiling_on_sc=False),
        scratch_types=[
            pltpu.VMEM((dpw * L,), jnp.int32),
            pltpu.VMEM((L, D), jnp.float32),
            pltpu.VMEM((L, D), jnp.float32),
            pltpu.VMEM((dpw, D), jnp.float32),
            pltpu.SemaphoreType.DMA,
            pltpu.SemaphoreType.DMA,
        ],
    )
    def sc_sum(tokens_hbm, table_hbm, out_hbm, idx_v, buf_a, buf_b, outblk,
               sem_a, sem_b):
        wid = lax.axis_index("s") * NC + lax.axis_index("c")
        base_doc = wid * dpw

        pltpu.sync_copy(tokens_hbm.at[pl.ds(base_doc * L, dpw * L)], idx_v)

        def gather_start(d, buf, sem):
            off = d * L
            for c_off, c_sz in chunks:
                pltpu.async_copy(
                    table_hbm.at[idx_v.at[pl.ds(off + c_off, c_sz)]],
                    buf.at[pl.ds(c_off, c_sz)],
                    sem,
                )

        def gather_wait(buf, sem):
            for c_off, c_sz in chunks:
                pltpu.make_async_copy(
                    table_hbm.at[idx_v.at[pl.ds(c_off, c_sz)]],
                    buf.at[pl.ds(c_off, c_sz)],
                    sem,
                ).wait()

        UNROLL = 8
        assert L % UNROLL == 0

        def reduce_doc(buf, d):
            zero = jnp.zeros((16,), jnp.float32)

            def body(t0, accs):
                accs = list(accs)
                for j in range(UNROLL):
                    t = t0 * UNROLL + j
                    for g in range(n_groups):
                        accs[g] = accs[g] + buf[t, pl.ds(g * 16, 16)]
                return tuple(accs)

            accs = lax.fori_loop(0, L // UNROLL, body, (zero,) * n_groups)
            for g in range(n_groups):
                outblk[d, pl.ds(g * 16, 16)] = accs[g]

        gather_start(0, buf_a, sem_a)

        def pair_body(i, _):
            d0 = 2 * i
            gather_start(d0 + 1, buf_b, sem_b)
            gather_wait(buf_a, sem_a)
            reduce_doc(buf_a, d0)

            @pl.when(d0 + 2 < dpw)
            def _():
                gather_start(d0 + 2, buf_a, sem_a)

            gather_wait(buf_b, sem_b)
            reduce_doc(buf_b, d0 + 1)
            return 0

        lax.fori_loop(0, dpw // 2, pair_body, 0)

        pltpu.sync_copy(outblk, out_hbm.at[pl.ds(base_doc, dpw)])

    return sc_sum


def _linear_body(sums_ref, len_ref, w_ref, b_ref, out_ref):
    inv = 1.0 / jnp.maximum(len_ref[...], 1).astype(jnp.float32)
    emb = sums_ref[...] * inv
    out_ref[...] = (
        jnp.dot(emb, w_ref[...], preferred_element_type=jnp.float32) + b_ref[...]
    )


def _tc_linear(sums, lens2d, W, b2d, BLK=512):
    B, D = sums.shape
    NL = W.shape[1]
    return pl.pallas_call(
        _linear_body,
        grid=(B // BLK,),
        in_specs=[
            pl.BlockSpec((BLK, D), lambda i: (i, 0)),
            pl.BlockSpec((BLK, 1), lambda i: (i, 0)),
            pl.BlockSpec((D, NL), lambda i: (0, 0)),
            pl.BlockSpec((1, NL), lambda i: (0, 0)),
        ],
        out_specs=pl.BlockSpec((BLK, NL), lambda i: (i, 0)),
        out_shape=jax.ShapeDtypeStruct((B, NL), jnp.float32),
    )(sums, lens2d, W, b2d)


@jax.jit
def kernel(doc_token, doc_token_len, token_table, W, b):
    B, L = doc_token.shape
    V, D = token_table.shape
    sc_sum = _make_sc_sum(B, L, V, D)

    H = 524288
    tt = token_table.T
    table2 = _tc_transpose(tt, V, D, H)
    table_rm = table2.reshape(2 * H, D)

    tok = doc_token.reshape(-1).astype(jnp.int32)
    tokens = jnp.where(tok < H, 2 * tok, 2 * (tok - H) + 1)
    sums = sc_sum(tokens, table_rm)
    lens2d = doc_token_len.reshape(B, 1)
    b2d = b.reshape(1, -1)
    return _tc_linear(sums, lens2d, W, b2d)
